# initial kernel scaffold (unmeasured)
import jax
import jax.numpy as jnp
from jax import lax
from jax.experimental import pallas as pl
from jax.experimental.pallas import tpu as pltpu

N_DEV = 4
N_EXPERTS = 16
EPD = N_EXPERTS // N_DEV
CAP = 192
BLOCK = EPD * CAP


def _a2a_moe_a2a(sendbuf, w1, w2):
    _, _, D = sendbuf.shape

    def body(send_ref, w1_ref, w2_ref, out_ref, recv_ref, ybuf_ref,
             s1, r1, s2, r2, lsem):
        me = lax.axis_index("i")

        barrier = pltpu.get_barrier_semaphore()
        for j in range(1, N_DEV):
            pl.semaphore_signal(
                barrier, inc=1,
                device_id=((me + j) % N_DEV,),
                device_id_type=pl.DeviceIdType.MESH,
            )
        pl.semaphore_wait(barrier, N_DEV - 1)

        p1 = []
        for j in range(1, N_DEV):
            p = (me + j) % N_DEV
            rdma = pltpu.make_async_remote_copy(
                src_ref=send_ref.at[p],
                dst_ref=recv_ref.at[me],
                send_sem=s1.at[p],
                recv_sem=r1.at[me],
                device_id=(p,),
                device_id_type=pl.DeviceIdType.MESH,
            )
            rdma.start()
            p1.append(rdma)

        own = pltpu.make_async_copy(send_ref.at[me], recv_ref.at[me], lsem)
        own.start()
        own.wait()

        for j in range(1, N_DEV):
            s = (me + j) % N_DEV
            pltpu.make_async_remote_copy(
                src_ref=send_ref.at[s],
                dst_ref=recv_ref.at[s],
                send_sem=s1.at[s],
                recv_sem=r1.at[s],
                device_id=(s,),
                device_id_type=pl.DeviceIdType.MESH,
            ).wait_recv()

        for k in range(EPD):
            for s in range(N_DEV):
                xk = recv_ref[s, k * CAP:(k + 1) * CAP, :]
                h = jnp.dot(xk, w1_ref[k], preferred_element_type=jnp.float32)
                h = jnp.maximum(h, 0.0).astype(jnp.bfloat16)
                y = jnp.dot(h, w2_ref[k], preferred_element_type=jnp.float32)
                ybuf_ref[s, k * CAP:(k + 1) * CAP, :] = y.astype(jnp.bfloat16)

        p2 = []
        for j in range(1, N_DEV):
            p = (me + j) % N_DEV
            rdma = pltpu.make_async_remote_copy(
                src_ref=ybuf_ref.at[p],
                dst_ref=out_ref.at[me],
                send_sem=s2.at[p],
                recv_sem=r2.at[me],
                device_id=(p,),
                device_id_type=pl.DeviceIdType.MESH,
            )
            rdma.start()
            p2.append(rdma)

        own2 = pltpu.make_async_copy(ybuf_ref.at[me], out_ref.at[me], lsem)
        own2.start()
        own2.wait()

        for j in range(1, N_DEV):
            s = (me + j) % N_DEV
            pltpu.make_async_remote_copy(
                src_ref=ybuf_ref.at[s],
                dst_ref=out_ref.at[s],
                send_sem=s2.at[s],
                recv_sem=r2.at[s],
                device_id=(s,),
                device_id_type=pl.DeviceIdType.MESH,
            ).wait_recv()

        for rdma in p1 + p2:
            rdma.wait_send()

    return pl.pallas_call(
        body,
        out_shape=jax.ShapeDtypeStruct((N_DEV, BLOCK, D), jnp.bfloat16),
        in_specs=[pl.BlockSpec(memory_space=pltpu.VMEM)] * 3,
        out_specs=pl.BlockSpec(memory_space=pltpu.VMEM),
        scratch_shapes=[
            pltpu.VMEM((N_DEV, BLOCK, D), jnp.bfloat16),
            pltpu.VMEM((N_DEV, BLOCK, D), jnp.bfloat16),
            pltpu.SemaphoreType.DMA((N_DEV,)),
            pltpu.SemaphoreType.DMA((N_DEV,)),
            pltpu.SemaphoreType.DMA((N_DEV,)),
            pltpu.SemaphoreType.DMA((N_DEV,)),
            pltpu.SemaphoreType.DMA,
        ],
        compiler_params=pltpu.CompilerParams(collective_id=0),
    )(sendbuf, w1, w2)


def kernel(x, assign, W1, W2):
    T, D = x.shape

    xb = x.astype(jnp.bfloat16)
    w1 = W1.astype(jnp.bfloat16)
    w2 = W2.astype(jnp.bfloat16)

    e = assign.astype(jnp.int32)
    oh = (e[:, None] == jnp.arange(N_EXPERTS, dtype=jnp.int32)[None, :])
    rank = jnp.take_along_axis(
        jnp.cumsum(oh.astype(jnp.int32), axis=0), e[:, None], axis=1
    )[:, 0] - 1
    slot = e * CAP + rank

    sendbuf = (
        jnp.zeros((N_EXPERTS * CAP, D), jnp.bfloat16)
        .at[slot].set(xb, mode="drop")
        .reshape(N_DEV, BLOCK, D)
    )

    outblocks = _a2a_moe_a2a(sendbuf, w1, w2)
    outflat = outblocks.reshape(N_EXPERTS * CAP, D)
    return jnp.take(outflat, slot, axis=0).astype(jnp.float32)


# baseline (device time: 193251 ns/iter reference)
import jax
import jax.numpy as jnp
from jax import lax
from jax.experimental import pallas as pl
from jax.experimental.pallas import tpu as pltpu

N_DEV = 4
N_EXPERTS = 16
EPD = N_EXPERTS // N_DEV
CAP = 192
BLOCK = EPD * CAP


def _a2a_moe_a2a(sendbuf, w1, w2):
    _, _, D = sendbuf.shape

    def body(send_ref, w1_ref, w2_ref, out_ref, recv_ref, ybuf_ref,
             s1, r1, s2, r2, lsem):
        me = lax.axis_index("i")

        barrier = pltpu.get_barrier_semaphore()
        for j in range(1, N_DEV):
            pl.semaphore_signal(
                barrier, inc=1,
                device_id=((me + j) % N_DEV,),
                device_id_type=pl.DeviceIdType.MESH,
            )
        pl.semaphore_wait(barrier, N_DEV - 1)

        p1 = []
        for j in range(1, N_DEV):
            p = (me + j) % N_DEV
            rdma = pltpu.make_async_remote_copy(
                src_ref=send_ref.at[p],
                dst_ref=recv_ref.at[me],
                send_sem=s1.at[p],
                recv_sem=r1.at[me],
                device_id=(p,),
                device_id_type=pl.DeviceIdType.MESH,
            )
            rdma.start()
            p1.append(rdma)

        own = pltpu.make_async_copy(send_ref.at[me], recv_ref.at[me], lsem)
        own.start()
        own.wait()

        for j in range(1, N_DEV):
            s = (me + j) % N_DEV
            pltpu.make_async_remote_copy(
                src_ref=send_ref.at[s],
                dst_ref=recv_ref.at[s],
                send_sem=s1.at[s],
                recv_sem=r1.at[s],
                device_id=(s,),
                device_id_type=pl.DeviceIdType.MESH,
            ).wait_recv()

        for k in range(EPD):
            for s in range(N_DEV):
                xk = recv_ref[s, k * CAP:(k + 1) * CAP, :]
                h = jnp.dot(xk, w1_ref[k], preferred_element_type=jnp.float32)
                h = jnp.maximum(h, 0.0).astype(jnp.bfloat16)
                y = jnp.dot(h, w2_ref[k], preferred_element_type=jnp.float32)
                ybuf_ref[s, k * CAP:(k + 1) * CAP, :] = y.astype(jnp.bfloat16)

        p2 = []
        for j in range(1, N_DEV):
            p = (me + j) % N_DEV
            rdma = pltpu.make_async_remote_copy(
                src_ref=ybuf_ref.at[p],
                dst_ref=out_ref.at[me],
                send_sem=s2.at[p],
                recv_sem=r2.at[me],
                device_id=(p,),
                device_id_type=pl.DeviceIdType.MESH,
            )
            rdma.start()
            p2.append(rdma)

        own2 = pltpu.make_async_copy(ybuf_ref.at[me], out_ref.at[me], lsem)
        own2.start()
        own2.wait()

        for j in range(1, N_DEV):
            s = (me + j) % N_DEV
            pltpu.make_async_remote_copy(
                src_ref=ybuf_ref.at[s],
                dst_ref=out_ref.at[s],
                send_sem=s2.at[s],
                recv_sem=r2.at[s],
                device_id=(s,),
                device_id_type=pl.DeviceIdType.MESH,
            ).wait_recv()

        for rdma in p1 + p2:
            rdma.wait_send()

    return pl.pallas_call(
        body,
        out_shape=jax.ShapeDtypeStruct((N_DEV, BLOCK, D), jnp.bfloat16),
        in_specs=[pl.BlockSpec(memory_space=pltpu.VMEM)] * 3,
        out_specs=pl.BlockSpec(memory_space=pltpu.VMEM),
        scratch_shapes=[
            pltpu.VMEM((N_DEV, BLOCK, D), jnp.bfloat16),
            pltpu.VMEM((N_DEV, BLOCK, D), jnp.bfloat16),
            pltpu.SemaphoreType.DMA((N_DEV,)),
            pltpu.SemaphoreType.DMA((N_DEV,)),
            pltpu.SemaphoreType.DMA((N_DEV,)),
            pltpu.SemaphoreType.DMA((N_DEV,)),
            pltpu.SemaphoreType.DMA,
        ],
        compiler_params=pltpu.CompilerParams(
            collective_id=0,
            vmem_limit_bytes=60 * 1024 * 1024,
        ),
    )(sendbuf, w1, w2)


def kernel(x, assign, W1, W2):
    T, D = x.shape

    xb = x.astype(jnp.bfloat16)
    w1 = W1.astype(jnp.bfloat16)
    w2 = W2.astype(jnp.bfloat16)

    e = assign.astype(jnp.int32)
    oh = (e[:, None] == jnp.arange(N_EXPERTS, dtype=jnp.int32)[None, :])
    rank = jnp.take_along_axis(
        jnp.cumsum(oh.astype(jnp.int32), axis=0), e[:, None], axis=1
    )[:, 0] - 1
    slot = e * CAP + rank

    sendbuf = (
        jnp.zeros((N_EXPERTS * CAP, D), jnp.bfloat16)
        .at[slot].set(xb, mode="drop")
        .reshape(N_DEV, BLOCK, D)
    )

    outblocks = _a2a_moe_a2a(sendbuf, w1, w2)
    outflat = outblocks.reshape(N_EXPERTS * CAP, D)
    return jnp.take(outflat, slot, axis=0).astype(jnp.float32)


# device time: 181657 ns/iter; 1.0638x vs baseline; 1.0638x over previous
import jax
import jax.numpy as jnp
from jax import lax
from jax.experimental import pallas as pl
from jax.experimental.pallas import tpu as pltpu

N_DEV = 4
N_EXPERTS = 16
EPD = N_EXPERTS // N_DEV
CAP = 160
BLOCK = EPD * CAP


def _cast_weights(W1, W2):
    _, D, F = W1.shape

    def body(w1_ref, w2_ref, o1_ref, o2_ref):
        o1_ref[...] = w1_ref[...].astype(jnp.bfloat16)
        o2_ref[...] = w2_ref[...].astype(jnp.bfloat16)

    return pl.pallas_call(
        body,
        grid=(EPD, 2),
        in_specs=[
            pl.BlockSpec((1, D, F // 2), lambda k, j: (k, 0, j)),
            pl.BlockSpec((1, F // 2, D), lambda k, j: (k, j, 0)),
        ],
        out_specs=[
            pl.BlockSpec((1, D, F // 2), lambda k, j: (k, 0, j)),
            pl.BlockSpec((1, F // 2, D), lambda k, j: (k, j, 0)),
        ],
        out_shape=[
            jax.ShapeDtypeStruct(W1.shape, jnp.bfloat16),
            jax.ShapeDtypeStruct(W2.shape, jnp.bfloat16),
        ],
    )(W1, W2)


def _a2a_moe_a2a(xb, slot_row, slot_col, w1, w2):
    T, D = xb.shape

    def body(x_ref, srow_ref, scol_ref, w1_ref, w2_ref, out_ref,
             send_ref, recv_ref, s1, r1, s2, r2, lsem):
        me = lax.axis_index("i")

        barrier = pltpu.get_barrier_semaphore()
        for j in range(1, N_DEV):
            pl.semaphore_signal(
                barrier, inc=1,
                device_id=((me + j) % N_DEV,),
                device_id_type=pl.DeviceIdType.MESH,
            )
        pl.semaphore_wait(barrier, N_DEV - 1)

        iota_r = lax.broadcasted_iota(jnp.int32, (BLOCK, 1), 0)
        for d in range(N_DEV):
            sd = (srow_ref[...] - d * BLOCK == iota_r).astype(jnp.bfloat16)
            send_ref[d, :, :] = jnp.dot(
                sd, x_ref[...], preferred_element_type=jnp.float32
            ).astype(jnp.bfloat16)

        p1 = []
        for j in range(1, N_DEV):
            p = (me + j) % N_DEV
            rdma = pltpu.make_async_remote_copy(
                src_ref=send_ref.at[p],
                dst_ref=recv_ref.at[me],
                send_sem=s1.at[p],
                recv_sem=r1.at[me],
                device_id=(p,),
                device_id_type=pl.DeviceIdType.MESH,
            )
            rdma.start()
            p1.append(rdma)

        own = pltpu.make_async_copy(send_ref.at[me], recv_ref.at[me], lsem)
        own.start()
        own.wait()

        for j in range(1, N_DEV):
            s = (me + j) % N_DEV
            pltpu.make_async_remote_copy(
                src_ref=send_ref.at[s],
                dst_ref=recv_ref.at[s],
                send_sem=s1.at[s],
                recv_sem=r1.at[s],
                device_id=(s,),
                device_id_type=pl.DeviceIdType.MESH,
            ).wait_recv()

        for k in range(EPD):
            for s in range(N_DEV):
                xk = recv_ref[s, k * CAP:(k + 1) * CAP, :]
                h = jnp.dot(xk, w1_ref[k], preferred_element_type=jnp.float32)
                h = jnp.maximum(h, 0.0).astype(jnp.bfloat16)
                y = jnp.dot(h, w2_ref[k], preferred_element_type=jnp.float32)
                recv_ref[s, k * CAP:(k + 1) * CAP, :] = y.astype(jnp.bfloat16)

        p2 = []
        for j in range(1, N_DEV):
            p = (me + j) % N_DEV
            rdma = pltpu.make_async_remote_copy(
                src_ref=recv_ref.at[p],
                dst_ref=send_ref.at[me],
                send_sem=s2.at[p],
                recv_sem=r2.at[me],
                device_id=(p,),
                device_id_type=pl.DeviceIdType.MESH,
            )
            rdma.start()
            p2.append(rdma)

        own2 = pltpu.make_async_copy(recv_ref.at[me], send_ref.at[me], lsem)
        own2.start()
        own2.wait()

        for j in range(1, N_DEV):
            s = (me + j) % N_DEV
            pltpu.make_async_remote_copy(
                src_ref=recv_ref.at[s],
                dst_ref=send_ref.at[s],
                send_sem=s2.at[s],
                recv_sem=r2.at[s],
                device_id=(s,),
                device_id_type=pl.DeviceIdType.MESH,
            ).wait_recv()

        iota_c = lax.broadcasted_iota(jnp.int32, (1, BLOCK), 1)
        H = T // 2
        for half in range(2):
            rows = slice(half * H, (half + 1) * H)
            acc = jnp.zeros((H, D), jnp.float32)
            for d in range(N_DEV):
                sd = (scol_ref[rows, :] - d * BLOCK == iota_c).astype(
                    jnp.bfloat16
                )
                acc = acc + jnp.dot(
                    sd, send_ref[d, :, :], preferred_element_type=jnp.float32
                )
            out_ref[rows, :] = acc

        for rdma in p1 + p2:
            rdma.wait_send()

    return pl.pallas_call(
        body,
        out_shape=jax.ShapeDtypeStruct((T, D), jnp.float32),
        in_specs=[pl.BlockSpec(memory_space=pltpu.VMEM)] * 5,
        out_specs=pl.BlockSpec(memory_space=pltpu.VMEM),
        scratch_shapes=[
            pltpu.VMEM((N_DEV, BLOCK, D), jnp.bfloat16),
            pltpu.VMEM((N_DEV, BLOCK, D), jnp.bfloat16),
            pltpu.SemaphoreType.DMA((N_DEV,)),
            pltpu.SemaphoreType.DMA((N_DEV,)),
            pltpu.SemaphoreType.DMA((N_DEV,)),
            pltpu.SemaphoreType.DMA((N_DEV,)),
            pltpu.SemaphoreType.DMA,
        ],
        compiler_params=pltpu.CompilerParams(
            collective_id=0,
            vmem_limit_bytes=63 * 1024 * 1024,
        ),
    )(xb, slot_row, slot_col, w1, w2)


def kernel(x, assign, W1, W2):
    T, _ = x.shape

    xb = x.astype(jnp.bfloat16)
    w1, w2 = _cast_weights(W1, W2)

    e = assign.astype(jnp.int32)
    oh = (e[:, None] == jnp.arange(N_EXPERTS, dtype=jnp.int32)[None, :])
    rank = jnp.take_along_axis(
        jnp.cumsum(oh.astype(jnp.int32), axis=0), e[:, None], axis=1
    )[:, 0] - 1
    slot = e * CAP + rank

    return _a2a_moe_a2a(
        xb, slot.reshape(1, T), slot.reshape(T, 1), w1, w2
    )


# device time: 164937 ns/iter; 1.1717x vs baseline; 1.1014x over previous
import jax
import jax.numpy as jnp
from jax import lax
from jax.experimental import pallas as pl
from jax.experimental.pallas import tpu as pltpu

N_DEV = 4
N_EXPERTS = 16
EPD = N_EXPERTS // N_DEV
CAP = 160
BLOCK = EPD * CAP


def _cast_weights(W1, W2):
    _, D, F = W1.shape

    def body(w1_ref, w2_ref, o1_ref, o2_ref):
        o1_ref[...] = w1_ref[...].astype(jnp.bfloat16)
        o2_ref[...] = w2_ref[...].astype(jnp.bfloat16)

    return pl.pallas_call(
        body,
        grid=(EPD, 2),
        in_specs=[
            pl.BlockSpec((1, D, F // 2), lambda k, j: (k, 0, j)),
            pl.BlockSpec((1, F // 2, D), lambda k, j: (k, j, 0)),
        ],
        out_specs=[
            pl.BlockSpec((1, D, F // 2), lambda k, j: (k, 0, j)),
            pl.BlockSpec((1, F // 2, D), lambda k, j: (k, j, 0)),
        ],
        out_shape=[
            jax.ShapeDtypeStruct(W1.shape, jnp.bfloat16),
            jax.ShapeDtypeStruct(W2.shape, jnp.bfloat16),
        ],
    )(W1, W2)


def _a2a_moe_a2a(xb, slot_row, slot_col, w1, w2):
    T, D = xb.shape

    def body(x_ref, srow_ref, scol_ref, w1_ref, w2_ref, out_ref,
             send_ref, recv_ref, s1, r1, s2, r2):
        me = lax.axis_index("i")

        barrier = pltpu.get_barrier_semaphore()
        for j in range(1, N_DEV):
            pl.semaphore_signal(
                barrier, inc=1,
                device_id=((me + j) % N_DEV,),
                device_id_type=pl.DeviceIdType.MESH,
            )

        iota_r = lax.broadcasted_iota(jnp.int32, (BLOCK, 1), 0)

        def pack(dst):
            sd = (srow_ref[...] - dst * BLOCK == iota_r).astype(jnp.bfloat16)
            return jnp.dot(
                sd, x_ref[...], preferred_element_type=jnp.float32
            ).astype(jnp.bfloat16)

        pl.semaphore_wait(barrier, N_DEV - 1)
        p1 = []
        for j in range(1, N_DEV):
            p = (me + j) % N_DEV
            send_ref[p] = pack(p)
            rdma = pltpu.make_async_remote_copy(
                src_ref=send_ref.at[p],
                dst_ref=recv_ref.at[me],
                send_sem=s1.at[p],
                recv_sem=r1.at[me],
                device_id=(p,),
                device_id_type=pl.DeviceIdType.MESH,
            )
            rdma.start()
            p1.append(rdma)

        recv_ref[me] = pack(me)

        p2 = []
        for j in range(N_DEV):
            s = (me + j) % N_DEV
            if j > 0:
                pltpu.make_async_remote_copy(
                    src_ref=send_ref.at[s],
                    dst_ref=recv_ref.at[s],
                    send_sem=s1.at[s],
                    recv_sem=r1.at[s],
                    device_id=(s,),
                    device_id_type=pl.DeviceIdType.MESH,
                ).wait_recv()
            for k in range(EPD):
                xk = recv_ref[s, k * CAP:(k + 1) * CAP, :]
                h = jnp.dot(xk, w1_ref[k], preferred_element_type=jnp.float32)
                h = jnp.maximum(h, 0.0).astype(jnp.bfloat16)
                y = jnp.dot(h, w2_ref[k], preferred_element_type=jnp.float32)
                recv_ref[s, k * CAP:(k + 1) * CAP, :] = y.astype(jnp.bfloat16)
            if j > 0:
                rdma = pltpu.make_async_remote_copy(
                    src_ref=recv_ref.at[s],
                    dst_ref=send_ref.at[me],
                    send_sem=s2.at[s],
                    recv_sem=r2.at[me],
                    device_id=(s,),
                    device_id_type=pl.DeviceIdType.MESH,
                )
                rdma.start()
                p2.append(rdma)

        iota_c = lax.broadcasted_iota(jnp.int32, (1, BLOCK), 1)
        H = T // 2
        for j in range(N_DEV):
            d = (me + j) % N_DEV
            if j > 0:
                pltpu.make_async_remote_copy(
                    src_ref=recv_ref.at[d],
                    dst_ref=send_ref.at[d],
                    send_sem=s2.at[d],
                    recv_sem=r2.at[d],
                    device_id=(d,),
                    device_id_type=pl.DeviceIdType.MESH,
                ).wait_recv()
            blk = recv_ref[d] if j == 0 else send_ref[d]
            for half in range(2):
                lo = half * H
                sd = (scol_ref[lo:lo + H, :] - d * BLOCK == iota_c).astype(
                    jnp.bfloat16
                )
                contrib = jnp.dot(
                    sd, blk, preferred_element_type=jnp.float32
                ).astype(jnp.bfloat16)
                if j == 0:
                    out_ref[lo:lo + H, :] = contrib
                else:
                    out_ref[lo:lo + H, :] = out_ref[lo:lo + H, :] + contrib

        for rdma in p1 + p2:
            rdma.wait_send()

    return pl.pallas_call(
        body,
        out_shape=jax.ShapeDtypeStruct((T, D), jnp.bfloat16),
        in_specs=[pl.BlockSpec(memory_space=pltpu.VMEM)] * 5,
        out_specs=pl.BlockSpec(memory_space=pltpu.VMEM),
        scratch_shapes=[
            pltpu.VMEM((N_DEV, BLOCK, D), jnp.bfloat16),
            pltpu.VMEM((N_DEV, BLOCK, D), jnp.bfloat16),
            pltpu.SemaphoreType.DMA((N_DEV,)),
            pltpu.SemaphoreType.DMA((N_DEV,)),
            pltpu.SemaphoreType.DMA((N_DEV,)),
            pltpu.SemaphoreType.DMA((N_DEV,)),
        ],
        compiler_params=pltpu.CompilerParams(
            collective_id=0,
            vmem_limit_bytes=63 * 1024 * 1024,
        ),
    )(xb, slot_row, slot_col, w1, w2)


def kernel(x, assign, W1, W2):
    T, _ = x.shape

    xb = x.astype(jnp.bfloat16)
    w1, w2 = _cast_weights(W1, W2)

    e = assign.astype(jnp.int32)
    oh = (e[:, None] == jnp.arange(N_EXPERTS, dtype=jnp.int32)[None, :])
    ohi = oh.astype(jnp.int32)
    rank = jnp.sum(jnp.cumsum(ohi, axis=0) * ohi, axis=1) - 1
    slot = e * CAP + rank

    out = _a2a_moe_a2a(
        xb, slot.reshape(1, T), slot.reshape(T, 1), w1, w2
    )
    return out.astype(jnp.float32)


# device time: 122313 ns/iter; 1.5800x vs baseline; 1.3485x over previous
import jax
import jax.numpy as jnp
from jax import lax
from jax.experimental import pallas as pl
from jax.experimental.pallas import tpu as pltpu

N_DEV = 4
N_EXPERTS = 16
EPD = N_EXPERTS // N_DEV
CAP = 160
BLOCK = EPD * CAP


def _a2a_moe_a2a(xb, slot_row, slot_col, W1, W2):
    T, D = xb.shape
    F = W1.shape[2]

    def body(x_ref, srow_ref, scol_ref, w1_hbm, w2_hbm, out_ref,
             send_ref, recv_ref, w1b_ref, w2b_ref, stage_ref,
             s1, r1, s2, r2, csem):
        me = lax.axis_index("i")

        barrier = pltpu.get_barrier_semaphore()
        for j in range(1, N_DEV):
            pl.semaphore_signal(
                barrier, inc=1,
                device_id=((me + j) % N_DEV,),
                device_id_type=pl.DeviceIdType.MESH,
            )

        iota_r = lax.broadcasted_iota(jnp.int32, (BLOCK, 1), 0)

        def pack(dst):
            sd = (srow_ref[...] - dst * BLOCK == iota_r).astype(jnp.bfloat16)
            return jnp.dot(
                sd, x_ref[...], preferred_element_type=jnp.float32
            ).astype(jnp.bfloat16)

        pl.semaphore_wait(barrier, N_DEV - 1)
        p1 = []
        for j in range(1, N_DEV):
            p = (me + j) % N_DEV
            send_ref[p] = pack(p)
            rdma = pltpu.make_async_remote_copy(
                src_ref=send_ref.at[p],
                dst_ref=recv_ref.at[me],
                send_sem=s1.at[p],
                recv_sem=r1.at[me],
                device_id=(p,),
                device_id_type=pl.DeviceIdType.MESH,
            )
            rdma.start()
            p1.append(rdma)

        recv_ref[me] = pack(me)

        n_chunks = EPD * 2

        def make_cast_loop(w_hbm, wb_ref, w1_layout):
            def chunk_copy(i):
                k = i // 2
                hs = pl.ds((i % 2) * (F // 2), F // 2)
                src = w_hbm.at[k, :, hs] if w1_layout else w_hbm.at[k, hs, :]
                return pltpu.make_async_copy(src, stage_ref.at[i % 2],
                                             csem.at[i % 2])

            def step(i, carry):
                @pl.when(i + 1 < n_chunks)
                def _():
                    chunk_copy(i + 1).start()
                chunk_copy(i).wait()
                k = i // 2
                hs = pl.ds((i % 2) * (F // 2), F // 2)
                val = stage_ref[i % 2].astype(jnp.bfloat16)
                if w1_layout:
                    wb_ref[k, :, hs] = val
                else:
                    wb_ref[k, hs, :] = val
                return carry

            chunk_copy(0).start()
            lax.fori_loop(0, n_chunks, step, 0)

        make_cast_loop(w1_hbm, w1b_ref, True)
        make_cast_loop(w2_hbm, w2b_ref, False)

        p2 = []
        for j in range(N_DEV):
            s = (me + j) % N_DEV
            if j > 0:
                pltpu.make_async_remote_copy(
                    src_ref=send_ref.at[s],
                    dst_ref=recv_ref.at[s],
                    send_sem=s1.at[s],
                    recv_sem=r1.at[s],
                    device_id=(s,),
                    device_id_type=pl.DeviceIdType.MESH,
                ).wait_recv()
            for k in range(EPD):
                xk = recv_ref[s, k * CAP:(k + 1) * CAP, :]
                h = jnp.dot(xk, w1b_ref[k],
                            preferred_element_type=jnp.float32)
                h = jnp.maximum(h, 0.0).astype(jnp.bfloat16)
                y = jnp.dot(h, w2b_ref[k],
                            preferred_element_type=jnp.float32)
                recv_ref[s, k * CAP:(k + 1) * CAP, :] = y.astype(jnp.bfloat16)
            if j > 0:
                rdma = pltpu.make_async_remote_copy(
                    src_ref=recv_ref.at[s],
                    dst_ref=send_ref.at[me],
                    send_sem=s2.at[s],
                    recv_sem=r2.at[me],
                    device_id=(s,),
                    device_id_type=pl.DeviceIdType.MESH,
                )
                rdma.start()
                p2.append(rdma)

        iota_c = lax.broadcasted_iota(jnp.int32, (1, BLOCK), 1)
        H = T // 2
        for j in range(N_DEV):
            d = (me + j) % N_DEV
            if j > 0:
                pltpu.make_async_remote_copy(
                    src_ref=recv_ref.at[d],
                    dst_ref=send_ref.at[d],
                    send_sem=s2.at[d],
                    recv_sem=r2.at[d],
                    device_id=(d,),
                    device_id_type=pl.DeviceIdType.MESH,
                ).wait_recv()
            blk = recv_ref[d] if j == 0 else send_ref[d]
            for half in range(2):
                lo = half * H
                sd = (scol_ref[lo:lo + H, :] - d * BLOCK == iota_c).astype(
                    jnp.bfloat16
                )
                contrib = jnp.dot(
                    sd, blk, preferred_element_type=jnp.float32
                ).astype(jnp.bfloat16)
                if j == 0:
                    out_ref[lo:lo + H, :] = contrib
                else:
                    out_ref[lo:lo + H, :] = out_ref[lo:lo + H, :] + contrib

        for rdma in p1 + p2:
            rdma.wait_send()

    return pl.pallas_call(
        body,
        out_shape=jax.ShapeDtypeStruct((T, D), jnp.bfloat16),
        in_specs=[
            pl.BlockSpec(memory_space=pltpu.MemorySpace.VMEM),
            pl.BlockSpec(memory_space=pltpu.MemorySpace.VMEM),
            pl.BlockSpec(memory_space=pltpu.MemorySpace.VMEM),
            pl.BlockSpec(memory_space=pltpu.MemorySpace.HBM),
            pl.BlockSpec(memory_space=pltpu.MemorySpace.HBM),
        ],
        out_specs=pl.BlockSpec(memory_space=pltpu.MemorySpace.VMEM),
        scratch_shapes=[
            pltpu.VMEM((N_DEV, BLOCK, D), jnp.bfloat16),
            pltpu.VMEM((N_DEV, BLOCK, D), jnp.bfloat16),
            pltpu.VMEM((EPD, D, F), jnp.bfloat16),
            pltpu.VMEM((EPD, F, D), jnp.bfloat16),
            pltpu.VMEM((2, D, D), jnp.float32),
            pltpu.SemaphoreType.DMA((N_DEV,)),
            pltpu.SemaphoreType.DMA((N_DEV,)),
            pltpu.SemaphoreType.DMA((N_DEV,)),
            pltpu.SemaphoreType.DMA((N_DEV,)),
            pltpu.SemaphoreType.DMA((2,)),
        ],
        compiler_params=pltpu.CompilerParams(
            collective_id=0,
            vmem_limit_bytes=63 * 1024 * 1024,
        ),
    )(xb, slot_row, slot_col, W1, W2)


def kernel(x, assign, W1, W2):
    T, _ = x.shape

    xb = x.astype(jnp.bfloat16)

    e = assign.astype(jnp.int32)
    oh = (e[:, None] == jnp.arange(N_EXPERTS, dtype=jnp.int32)[None, :])
    ohi = oh.astype(jnp.int32)
    rank = jnp.sum(jnp.cumsum(ohi, axis=0) * ohi, axis=1) - 1
    slot = e * CAP + rank

    out = _a2a_moe_a2a(
        xb, slot.reshape(1, T), slot.reshape(T, 1), W1, W2
    )
    return out.astype(jnp.float32)
